# Initial kernel scaffold; baseline (speedup 1.0000x reference)
import jax
import jax.numpy as jnp
from jax.experimental import pallas as pl


def kernel(x, edge_index, edge_type, W1, Ws1, b1, W2, Ws2, b2, W3, Ws3, b3):
    raise NotImplementedError("write your pallas kernel here")


# speedup vs baseline: 9.3249x; 9.3249x over previous
import functools

import jax
import jax.numpy as jnp
from jax import lax
from jax.experimental import pallas as pl
from jax.experimental.pallas import tpu as pltpu
from jax.experimental.pallas import tpu_sc as plsc

N = 10000
E = 320000
R = 8
H = 128

NP = 10112
ROWS_PER_TILE = NP // 16
DUMMY_DST = 10008

NTILES = 32
EPAD = 327680
EPW = EPAD // NTILES
CH = 128
NCHUNK = EPW // CH

BN = 400
NB = N // BN


def _mm_body(x_ref, w_ref, o_ref):
    xb = x_ref[...]
    for r in range(9):
        o_ref[r] = jnp.dot(xb, w_ref[r], preferred_element_type=jnp.float32)


def _layer1_matmul(x, wc):
    return pl.pallas_call(
        _mm_body,
        grid=(NB,),
        in_specs=[
            pl.BlockSpec((BN, H), lambda i: (i, 0)),
            pl.BlockSpec((9, H, H), lambda i: (0, 0, 0)),
        ],
        out_specs=pl.BlockSpec((9, BN, H), lambda i: (0, i, 0)),
        out_shape=jax.ShapeDtypeStruct((9, N, H), jnp.float32),
    )(x, wc)


def _mmc_body(a_ref, s_ref, b_ref, w_ref, o_ref, h_ref):
    hb = jnp.maximum(a_ref[0] + a_ref[1] + s_ref[0] + b_ref[...], 0.0)
    h_ref[...] = hb
    for r in range(9):
        o_ref[r] = jnp.dot(hb, w_ref[r], preferred_element_type=jnp.float32)


def _layer_combine_matmul(agg, hw_prev, b, wc):
    return pl.pallas_call(
        _mmc_body,
        grid=(NB,),
        in_specs=[
            pl.BlockSpec((2, BN, H), lambda i: (0, i, 0)),
            pl.BlockSpec((1, BN, H), lambda i: (8, i, 0)),
            pl.BlockSpec((1, H), lambda i: (0, 0)),
            pl.BlockSpec((9, H, H), lambda i: (0, 0, 0)),
        ],
        out_specs=[
            pl.BlockSpec((9, BN, H), lambda i: (0, i, 0)),
            pl.BlockSpec((BN, H), lambda i: (i, 0)),
        ],
        out_shape=[
            jax.ShapeDtypeStruct((9, N, H), jnp.float32),
            jax.ShapeDtypeStruct((N, H), jnp.float32),
        ],
    )(agg, hw_prev, b, wc)


def _final_body(h1_ref, h2_ref, a_ref, s_ref, b_ref, o_ref):
    h3 = jnp.maximum(a_ref[0] + a_ref[1] + s_ref[0] + b_ref[...], 0.0)
    row = jnp.concatenate(
        [
            jnp.sum(h1_ref[...], axis=0, keepdims=True),
            jnp.sum(h2_ref[...], axis=0, keepdims=True),
            jnp.sum(h3, axis=0, keepdims=True),
        ],
        axis=1,
    )
    i = pl.program_id(0)

    @pl.when(i == 0)
    def _():
        o_ref[...] = row

    @pl.when(i != 0)
    def _():
        o_ref[...] += row


def _final_pool(h1, h2, agg, hw_prev, b):
    return pl.pallas_call(
        _final_body,
        grid=(NB,),
        in_specs=[
            pl.BlockSpec((BN, H), lambda i: (i, 0)),
            pl.BlockSpec((BN, H), lambda i: (i, 0)),
            pl.BlockSpec((2, BN, H), lambda i: (0, i, 0)),
            pl.BlockSpec((1, BN, H), lambda i: (8, i, 0)),
            pl.BlockSpec((1, H), lambda i: (0, 0)),
        ],
        out_specs=pl.BlockSpec((1, 3 * H), lambda i: (0, 0)),
        out_shape=jax.ShapeDtypeStruct((1, 3 * H), jnp.float32),
    )(h1, h2, agg, hw_prev, b)


def _edge_kernel(hw_hbm, gidx_hbm, dst_hbm, zeros_hbm, out_hbm,
                 gidx_v, dst_v, rows_v, acc_sh, sem):
    cid = lax.axis_index("c")
    sid = lax.axis_index("s")
    tid = cid * 16 + sid

    pltpu.sync_copy(zeros_hbm.at[pl.ds(sid * ROWS_PER_TILE, ROWS_PER_TILE)],
                    acc_sh.at[pl.ds(sid * ROWS_PER_TILE, ROWS_PER_TILE)])
    plsc.subcore_barrier()

    def step(i, carry):
        base = tid * EPW + i * CH
        pltpu.sync_copy(gidx_hbm.at[pl.ds(base, CH)], gidx_v)
        pltpu.sync_copy(dst_hbm.at[pl.ds(base, CH)], dst_v)
        pltpu.async_copy(hw_hbm.at[gidx_v], rows_v, sem).wait()
        pltpu.sync_copy(rows_v, acc_sh.at[dst_v], add=True)
        return carry

    lax.fori_loop(0, NCHUNK, step, 0)
    plsc.subcore_barrier()

    pltpu.sync_copy(acc_sh.at[pl.ds(sid * ROWS_PER_TILE, ROWS_PER_TILE)],
                    out_hbm.at[cid, pl.ds(sid * ROWS_PER_TILE, ROWS_PER_TILE)])


def _edge_aggregate(hw_flat, gidx, dst, zeros):
    mesh = plsc.VectorSubcoreMesh(core_axis_name="c", subcore_axis_name="s")
    return pl.kernel(
        _edge_kernel,
        mesh=mesh,
        out_type=jax.ShapeDtypeStruct((2, NP, H), jnp.float32),
        scratch_types=[
            pltpu.VMEM((CH,), jnp.int32),
            pltpu.VMEM((CH,), jnp.int32),
            pltpu.VMEM((CH, H), jnp.float32),
            pltpu.VMEM_SHARED((NP, H), jnp.float32),
            pltpu.SemaphoreType.DMA,
        ],
    )(hw_flat, gidx, dst, zeros)


def kernel(x, edge_index, edge_type, W1, Ws1, b1, W2, Ws2, b2, W3, Ws3, b3):
    src = edge_index[0]
    dst = edge_index[1]

    gidx = edge_type * N + src
    npad = EPAD - E
    gidx_p = jnp.concatenate([gidx, jnp.zeros((npad,), jnp.int32)])
    dst_p = jnp.concatenate([dst, jnp.full((npad,), DUMMY_DST, jnp.int32)])
    zeros = jnp.zeros((NP, H), jnp.float32)

    wc1 = jnp.concatenate([W1, Ws1[None]], axis=0)
    wc2 = jnp.concatenate([W2, Ws2[None]], axis=0)
    wc3 = jnp.concatenate([W3, Ws3[None]], axis=0)

    hw1 = _layer1_matmul(x, wc1)
    agg1 = _edge_aggregate(hw1.reshape(9 * N, H), gidx_p, dst_p, zeros)

    hw2, h1 = _layer_combine_matmul(agg1, hw1, b1.reshape(1, H), wc2)
    agg2 = _edge_aggregate(hw2.reshape(9 * N, H), gidx_p, dst_p, zeros)

    hw3, h2 = _layer_combine_matmul(agg2, hw2, b2.reshape(1, H), wc3)
    agg3 = _edge_aggregate(hw3.reshape(9 * N, H), gidx_p, dst_p, zeros)

    out = _final_pool(h1, h2, agg3, hw3, b3.reshape(1, H))
    return out.reshape(1, 1, 3 * H)


# speedup vs baseline: 11.4835x; 1.2315x over previous
import jax
import jax.numpy as jnp
from jax import lax
from jax.experimental import pallas as pl
from jax.experimental.pallas import tpu as pltpu
from jax.experimental.pallas import tpu_sc as plsc

N = 10000
E = 320000
H = 128

NP = 10112
ROWS_PER_TILE = NP // 16
DUMMY_DST = 10008

NTILES = 32
EPAD = 327680
EPW = EPAD // NTILES
CH = 64
NCHUNK = EPW // CH

NSLOT = 5
ILAG = 3
GLAG = 2
NG = NCHUNK // NSLOT

BN = 400
NB = N // BN


def _mm_body(x_ref, w_ref, o_ref):
    xb = x_ref[...]
    for r in range(9):
        o_ref[r] = jnp.dot(xb, w_ref[r], preferred_element_type=jnp.float32)


def _layer1_matmul(x, wc):
    return pl.pallas_call(
        _mm_body,
        grid=(NB,),
        in_specs=[
            pl.BlockSpec((BN, H), lambda i: (i, 0)),
            pl.BlockSpec((9, H, H), lambda i: (0, 0, 0)),
        ],
        out_specs=pl.BlockSpec((9, BN, H), lambda i: (0, i, 0)),
        out_shape=jax.ShapeDtypeStruct((9, N, H), jnp.float32),
    )(x, wc)


def _mmc_body(a_ref, s_ref, b_ref, w_ref, o_ref, h_ref):
    hb = jnp.maximum(a_ref[0] + a_ref[1] + s_ref[0] + b_ref[...], 0.0)
    h_ref[...] = hb
    for r in range(9):
        o_ref[r] = jnp.dot(hb, w_ref[r], preferred_element_type=jnp.float32)


def _layer_combine_matmul(agg, hw_prev, b, wc):
    return pl.pallas_call(
        _mmc_body,
        grid=(NB,),
        in_specs=[
            pl.BlockSpec((2, BN, H), lambda i: (0, i, 0)),
            pl.BlockSpec((1, BN, H), lambda i: (8, i, 0)),
            pl.BlockSpec((1, H), lambda i: (0, 0)),
            pl.BlockSpec((9, H, H), lambda i: (0, 0, 0)),
        ],
        out_specs=[
            pl.BlockSpec((9, BN, H), lambda i: (0, i, 0)),
            pl.BlockSpec((BN, H), lambda i: (i, 0)),
        ],
        out_shape=[
            jax.ShapeDtypeStruct((9, N, H), jnp.float32),
            jax.ShapeDtypeStruct((N, H), jnp.float32),
        ],
    )(agg, hw_prev, b, wc)


def _final_body(h1_ref, h2_ref, a_ref, s_ref, b_ref, o_ref):
    h3 = jnp.maximum(a_ref[0] + a_ref[1] + s_ref[0] + b_ref[...], 0.0)
    row = jnp.concatenate(
        [
            jnp.sum(h1_ref[...], axis=0, keepdims=True),
            jnp.sum(h2_ref[...], axis=0, keepdims=True),
            jnp.sum(h3, axis=0, keepdims=True),
        ],
        axis=1,
    )
    i = pl.program_id(0)

    @pl.when(i == 0)
    def _():
        o_ref[...] = row

    @pl.when(i != 0)
    def _():
        o_ref[...] += row


def _final_pool(h1, h2, agg, hw_prev, b):
    return pl.pallas_call(
        _final_body,
        grid=(NB,),
        in_specs=[
            pl.BlockSpec((BN, H), lambda i: (i, 0)),
            pl.BlockSpec((BN, H), lambda i: (i, 0)),
            pl.BlockSpec((2, BN, H), lambda i: (0, i, 0)),
            pl.BlockSpec((1, BN, H), lambda i: (8, i, 0)),
            pl.BlockSpec((1, H), lambda i: (0, 0)),
        ],
        out_specs=pl.BlockSpec((1, 3 * H), lambda i: (0, 0)),
        out_shape=jax.ShapeDtypeStruct((1, 3 * H), jnp.float32),
    )(h1, h2, agg, hw_prev, b)


def _edge_kernel(hw_hbm, gidx_hbm, dst_hbm, zeros_hbm, out_hbm,
                 gidx_r, dst_r, rows_v, acc_sh, isem, jsem, gsem, ssem, zsem):
    cid = lax.axis_index("c")
    sid = lax.axis_index("s")
    tid = cid * 16 + sid

    zcp = pltpu.make_async_copy(
        zeros_hbm.at[pl.ds(sid * ROWS_PER_TILE, ROWS_PER_TILE)],
        acc_sh.at[pl.ds(sid * ROWS_PER_TILE, ROWS_PER_TILE)], zsem)
    zcp.start()

    def start_idx(c):
        slot = c % NSLOT
        base = (tid * NCHUNK + c) * CH
        pltpu.make_async_copy(gidx_hbm.at[pl.ds(base, CH)],
                              gidx_r.at[slot], isem.at[slot]).start()
        pltpu.make_async_copy(dst_hbm.at[pl.ds(base, CH)],
                              dst_r.at[slot], jsem.at[slot]).start()

    def wait_idx(c):
        slot = c % NSLOT
        pltpu.make_async_copy(gidx_hbm.at[pl.ds(0, CH)],
                              gidx_r.at[slot], isem.at[slot]).wait()
        pltpu.make_async_copy(dst_hbm.at[pl.ds(0, CH)],
                              dst_r.at[slot], jsem.at[slot]).wait()

    def start_gather(c):
        slot = c % NSLOT
        pltpu.make_async_copy(hw_hbm.at[gidx_r.at[slot]],
                              rows_v.at[slot], gsem.at[slot]).start()

    def wait_gather(c):
        slot = c % NSLOT
        pltpu.make_async_copy(hw_hbm.at[gidx_r.at[slot]],
                              rows_v.at[slot], gsem.at[slot]).wait()

    def start_scatter(c):
        slot = c % NSLOT
        pltpu.async_copy(rows_v.at[slot], acc_sh.at[dst_r.at[slot]],
                         ssem.at[slot], add=True)

    def wait_scatter(c):
        slot = c % NSLOT
        pltpu.make_async_copy(rows_v.at[slot], acc_sh.at[dst_r.at[slot]],
                              ssem.at[slot]).wait()

    def steady(t, first=False, last=False):
        if not first or t >= NSLOT - ILAG:
            wait_scatter(t - (NSLOT - ILAG))
        if not last or t + ILAG < NCHUNK:
            start_idx(t + ILAG)
        if not last or t + GLAG < NCHUNK:
            wait_idx(t + GLAG)
            start_gather(t + GLAG)
        wait_gather(t)
        start_scatter(t)

    for c in range(ILAG):
        start_idx(c)
    for c in range(GLAG):
        wait_idx(c)
        start_gather(c)
    zcp.wait()
    plsc.subcore_barrier()

    for b in range(NSLOT):
        steady(b, first=True)

    def outer(G, carry):
        for b in range(NSLOT):
            t = G * NSLOT + b
            wait_scatter_slot = (b - (NSLOT - ILAG)) % NSLOT
            pltpu.make_async_copy(
                rows_v.at[wait_scatter_slot],
                acc_sh.at[dst_r.at[wait_scatter_slot]],
                ssem.at[wait_scatter_slot]).wait()
            islot = (b + ILAG) % NSLOT
            base = (tid * NCHUNK + t + ILAG) * CH
            pltpu.make_async_copy(gidx_hbm.at[pl.ds(base, CH)],
                                  gidx_r.at[islot], isem.at[islot]).start()
            pltpu.make_async_copy(dst_hbm.at[pl.ds(base, CH)],
                                  dst_r.at[islot], jsem.at[islot]).start()
            gslot = (b + GLAG) % NSLOT
            pltpu.make_async_copy(gidx_hbm.at[pl.ds(0, CH)],
                                  gidx_r.at[gslot], isem.at[gslot]).wait()
            pltpu.make_async_copy(dst_hbm.at[pl.ds(0, CH)],
                                  dst_r.at[gslot], jsem.at[gslot]).wait()
            pltpu.make_async_copy(hw_hbm.at[gidx_r.at[gslot]],
                                  rows_v.at[gslot], gsem.at[gslot]).start()
            pltpu.make_async_copy(hw_hbm.at[gidx_r.at[b]],
                                  rows_v.at[b], gsem.at[b]).wait()
            pltpu.async_copy(rows_v.at[b], acc_sh.at[dst_r.at[b]],
                             ssem.at[b], add=True)
        return carry

    lax.fori_loop(1, NG - 1, outer, 0)

    for b in range(NSLOT):
        steady((NG - 1) * NSLOT + b, last=True)

    for c in range(NCHUNK - (NSLOT - ILAG), NCHUNK):
        wait_scatter(c)

    plsc.subcore_barrier()

    pltpu.sync_copy(acc_sh.at[pl.ds(sid * ROWS_PER_TILE, ROWS_PER_TILE)],
                    out_hbm.at[cid, pl.ds(sid * ROWS_PER_TILE, ROWS_PER_TILE)])


def _edge_aggregate(hw_flat, gidx, dst, zeros):
    mesh = plsc.VectorSubcoreMesh(core_axis_name="c", subcore_axis_name="s")
    return pl.kernel(
        _edge_kernel,
        mesh=mesh,
        out_type=jax.ShapeDtypeStruct((2, NP, H), jnp.float32),
        scratch_types=[
            pltpu.VMEM((NSLOT, CH), jnp.int32),
            pltpu.VMEM((NSLOT, CH), jnp.int32),
            pltpu.VMEM((NSLOT, CH, H), jnp.float32),
            pltpu.VMEM_SHARED((NP, H), jnp.float32),
            pltpu.SemaphoreType.DMA((NSLOT,)),
            pltpu.SemaphoreType.DMA((NSLOT,)),
            pltpu.SemaphoreType.DMA((NSLOT,)),
            pltpu.SemaphoreType.DMA((NSLOT,)),
            pltpu.SemaphoreType.DMA,
        ],
    )(hw_flat, gidx, dst, zeros)


def kernel(x, edge_index, edge_type, W1, Ws1, b1, W2, Ws2, b2, W3, Ws3, b3):
    src = edge_index[0]
    dst = edge_index[1]

    gidx = edge_type * N + src
    npad = EPAD - E
    gidx_p = jnp.concatenate([gidx, jnp.zeros((npad,), jnp.int32)])
    dst_p = jnp.concatenate([dst, jnp.full((npad,), DUMMY_DST, jnp.int32)])
    zeros = jnp.zeros((NP, H), jnp.float32)

    wc1 = jnp.concatenate([W1, Ws1[None]], axis=0)
    wc2 = jnp.concatenate([W2, Ws2[None]], axis=0)
    wc3 = jnp.concatenate([W3, Ws3[None]], axis=0)

    hw1 = _layer1_matmul(x, wc1)
    agg1 = _edge_aggregate(hw1.reshape(9 * N, H), gidx_p, dst_p, zeros)

    hw2, h1 = _layer_combine_matmul(agg1, hw1, b1.reshape(1, H), wc2)
    agg2 = _edge_aggregate(hw2.reshape(9 * N, H), gidx_p, dst_p, zeros)

    hw3, h2 = _layer_combine_matmul(agg2, hw2, b2.reshape(1, H), wc3)
    agg3 = _edge_aggregate(hw3.reshape(9 * N, H), gidx_p, dst_p, zeros)

    out = _final_pool(h1, h2, agg3, hw3, b3.reshape(1, H))
    return out.reshape(1, 1, 3 * H)
